# Initial kernel scaffold; baseline (speedup 1.0000x reference)
import jax
import jax.numpy as jnp
from jax.experimental import pallas as pl


def kernel(predictions, targets):
    raise NotImplementedError("write your pallas kernel here")


# speedup vs baseline: 1.3567x; 1.3567x over previous
import jax
import jax.numpy as jnp
from jax import lax
from jax.experimental import pallas as pl
from jax.experimental.pallas import tpu as pltpu
from jax.experimental.pallas import tpu_sc as plsc

_S = 13
_CELLS = _S * _S
_NCH = 19
_CONF2 = 36
_T = 20
_L = 16
_NW = 32
_BATCH = 128
_BPW = _BATCH // _NW


def _body(preds_hbm, tg_hbm, out_hbm, tg_v, blk_v, c2_v, hit_v, acc_v):
    wid = lax.axis_index("s") * 2 + lax.axis_index("c")
    lanes = lax.iota(jnp.int32, _L)
    zeros = jnp.zeros((_L,), jnp.float32)
    ones = jnp.ones((_L,), jnp.float32)
    tail9 = lanes < (_CELLS - 10 * _L)
    tail_idx = jnp.minimum(lanes + 10 * _L, _CELLS - 1)

    acc_m = zeros
    acc_c = zeros

    for i in range(_BPW):
        b = wid * _BPW + i
        pltpu.sync_copy(tg_hbm.at[b], tg_v)
        pltpu.sync_copy(preds_hbm.at[b, 0:_NCH], blk_v)
        pltpu.sync_copy(preds_hbm.at[b, _CONF2], c2_v)

        for j in range(11):
            hit_v[pl.ds(j * _L, _L)] = zeros

        for chunk in range(2):
            tvec = lanes + chunk * _L
            fidx = jnp.minimum(tvec, _T - 1) * 5
            cls = plsc.load_gather(tg_v, [fidx])
            cx = plsc.load_gather(tg_v, [fidx + 1])
            cy = plsc.load_gather(tg_v, [fidx + 2])
            w = plsc.load_gather(tg_v, [fidx + 3])
            h = plsc.load_gather(tg_v, [fidx + 4])

            gx = (cx * _S).astype(jnp.int32)
            gy = (cy * _S).astype(jnp.int32)
            valid = (gx < _S) & (gy < _S) & (tvec < _T)
            gxc = jnp.clip(gx, 0, _S - 1)
            gyc = jnp.clip(gy, 0, _S - 1)
            cell = gyc * _S + gxc

            def pick(ch):
                return plsc.load_gather(
                    blk_v, [jnp.full((_L,), ch, jnp.int32), cell])

            d1 = pick(1) - cx
            d2 = pick(2) - cy
            d3 = pick(3) - w
            d4 = pick(4) - h
            coord = d1 * d1 + d2 * d2 + d3 * d3 + d4 * d4

            k = cls.astype(jnp.int32)
            cls_l = zeros
            for c in range(13):
                p = pick(5 + c)
                d = jnp.where(k == c, p - 1.0, p)
                cls_l = cls_l + d * d

            contrib = 5.0 * coord + cls_l
            acc_m = acc_m + jnp.where(valid, contrib, 0.0)

            plsc.store_scatter(hit_v, [cell], ones, mask=valid)

        for j in range(10):
            off = j * _L
            c0 = blk_v[0, pl.ds(off, _L)]
            c1 = blk_v[_NCH - 1, pl.ds(off, _L)]
            c2 = c2_v[pl.ds(off, _L)]
            hh = hit_v[pl.ds(off, _L)]
            acc_c = acc_c + (1.0 - hh) * c0 * c0 + c1 * c1 + c2 * c2
        zsplat = jnp.zeros((_L,), jnp.int32)
        c0 = plsc.load_gather(blk_v, [zsplat, tail_idx])
        c1 = plsc.load_gather(blk_v, [zsplat + (_NCH - 1), tail_idx])
        c2 = plsc.load_gather(c2_v, [tail_idx])
        hh = hit_v[pl.ds(10 * _L, _L)]
        c0 = jnp.where(tail9, c0, 0.0)
        c1 = jnp.where(tail9, c1, 0.0)
        c2 = jnp.where(tail9, c2, 0.0)
        acc_c = acc_c + (1.0 - hh) * c0 * c0 + c1 * c1 + c2 * c2

    acc_v[...] = acc_m + 0.5 * acc_c
    pltpu.sync_copy(acc_v, out_hbm.at[wid])


def kernel(predictions, targets):
    preds3 = predictions.reshape(_BATCH, 54, _CELLS)
    tg2 = targets.reshape(_BATCH, 5 * _T)
    mesh = plsc.VectorSubcoreMesh(
        core_axis_name="c", subcore_axis_name="s", num_cores=2, num_subcores=16)
    out = pl.kernel(
        _body,
        out_type=jax.ShapeDtypeStruct((_NW, _L), jnp.float32),
        mesh=mesh,
        compiler_params=pltpu.CompilerParams(
            use_tc_tiling_on_sc=False, needs_layout_passes=False),
        scratch_types=[
            pltpu.VMEM((5 * _T,), jnp.float32),
            pltpu.VMEM((_NCH, _CELLS), jnp.float32),
            pltpu.VMEM((_CELLS,), jnp.float32),
            pltpu.VMEM((176,), jnp.float32),
            pltpu.VMEM((_L,), jnp.float32),
        ],
    )(preds3, tg2)
    return jnp.sum(out) / _BATCH
